# Initial kernel scaffold; baseline (speedup 1.0000x reference)
import jax
import jax.numpy as jnp
from jax.experimental import pallas as pl


def kernel(x, adj, W1, b1, W2, b2, Wc, bc):
    raise NotImplementedError("write your pallas kernel here")


# speedup vs baseline: 12.5378x; 12.5378x over previous
import functools

import jax
import jax.numpy as jnp
from jax import lax
from jax.experimental import pallas as pl
from jax.experimental.pallas import tpu as pltpu
from jax.experimental.pallas import tpu_sc as plsc

N = 10000
E = 320000
F_IN = 128
H = 128
C = 40

NC = 2
NS = 16
NW = NC * NS
EPW = E // NW
CH = 80
NCHUNK = EPW // CH
NPAD = 640 * NS
RPT = 640
ZR = 128

_MESH = plsc.VectorSubcoreMesh(core_axis_name="c", subcore_axis_name="s")


def _sc_degree(dst, zeros640):
    @functools.partial(
        pl.kernel,
        out_type=jax.ShapeDtypeStruct((NC, NPAD), jnp.float32),
        mesh=_MESH,
        scratch_types=[
            pltpu.VMEM((CH,), jnp.int32),
            pltpu.VMEM((CH,), jnp.float32),
            pltpu.VMEM((640,), jnp.float32),
            pltpu.VMEM_SHARED((NPAD,), jnp.float32),
        ],
    )
    def k(dst_hbm, z_hbm, out_hbm, dst_v, ones_v, buf_v, deg_sh):
        c = lax.axis_index("c")
        s = lax.axis_index("s")
        wid = s * NC + c
        one16 = jnp.ones((16,), jnp.float32)
        for j in range(CH // 16):
            ones_v[pl.ds(j * 16, 16)] = one16
        pltpu.sync_copy(z_hbm, buf_v)
        pltpu.sync_copy(buf_v, deg_sh.at[pl.ds(pl.multiple_of(s * 640, 128), 640)])
        plsc.subcore_barrier()

        def body(g, carry):
            base = pl.multiple_of(wid * EPW + g * CH, 8)
            pltpu.sync_copy(dst_hbm.at[pl.ds(base, CH)], dst_v)
            pltpu.sync_copy(ones_v, deg_sh.at[dst_v], add=True)
            return carry

        lax.fori_loop(0, NCHUNK, body, 0)
        plsc.subcore_barrier()
        off = pl.multiple_of(s * 640, 128)
        pltpu.sync_copy(deg_sh.at[pl.ds(off, 640)], buf_v)
        pltpu.sync_copy(buf_v, out_hbm.at[c, pl.ds(off, 640)])

    return k(dst, zeros640)


def _sc_agg(xw, src, dst, zrows):
    @functools.partial(
        pl.kernel,
        out_type=jax.ShapeDtypeStruct((NC, NPAD, H), jnp.float32),
        mesh=_MESH,
        scratch_types=[
            pltpu.VMEM((CH,), jnp.int32),
            pltpu.VMEM((CH,), jnp.int32),
            pltpu.VMEM((CH, H), jnp.float32),
            pltpu.VMEM((ZR, H), jnp.float32),
            pltpu.VMEM_SHARED((NPAD, H), jnp.float32),
            pltpu.SemaphoreType.DMA,
        ],
    )
    def k(xw_hbm, src_hbm, dst_hbm, z_hbm, out_hbm,
          src_v, dst_v, rows_v, buf_v, acc_sh, sem):
        c = lax.axis_index("c")
        s = lax.axis_index("s")
        wid = s * NC + c
        pltpu.sync_copy(z_hbm, buf_v)
        for j in range(RPT // ZR):
            off = pl.multiple_of(s * RPT + j * ZR, 128)
            pltpu.sync_copy(buf_v, acc_sh.at[pl.ds(off, ZR)])
        plsc.subcore_barrier()

        def body(g, carry):
            base = pl.multiple_of(wid * EPW + g * CH, 8)
            pltpu.sync_copy(src_hbm.at[pl.ds(base, CH)], src_v)
            pltpu.sync_copy(dst_hbm.at[pl.ds(base, CH)], dst_v)
            pltpu.async_copy(xw_hbm.at[src_v], rows_v, sem).wait()
            pltpu.sync_copy(rows_v, acc_sh.at[dst_v], add=True)
            return carry

        lax.fori_loop(0, NCHUNK, body, 0)
        plsc.subcore_barrier()
        for j in range(RPT // ZR):
            off = pl.multiple_of(s * RPT + j * ZR, 128)
            pltpu.sync_copy(acc_sh.at[pl.ds(off, ZR)], buf_v)
            pltpu.sync_copy(buf_v, out_hbm.at[c, pl.ds(off, ZR)])

    return k(xw, src, dst, zrows)


_RB = 1000
_GRID = N // _RB


def _dis(d0, d1):
    return lax.rsqrt(d0 + d1 + 1.0)


_PREC = lax.Precision.HIGHEST


def _mm_scale_body(x_ref, w_ref, d0_ref, d1_ref, o_ref):
    dis = _dis(d0_ref[...], d1_ref[...])
    o_ref[...] = jnp.dot(x_ref[...], w_ref[...], precision=_PREC,
                         preferred_element_type=jnp.float32) * dis


def _tc_mm_scale(x, W, d0, d1):
    return pl.pallas_call(
        _mm_scale_body,
        grid=(_GRID,),
        in_specs=[
            pl.BlockSpec((_RB, F_IN), lambda i: (i, 0)),
            pl.BlockSpec((F_IN, H), lambda i: (0, 0)),
            pl.BlockSpec((_RB, 1), lambda i: (i, 0)),
            pl.BlockSpec((_RB, 1), lambda i: (i, 0)),
        ],
        out_specs=pl.BlockSpec((_RB, H), lambda i: (i, 0)),
        out_shape=jax.ShapeDtypeStruct((N, H), jnp.float32),
    )(x, W, d0, d1)


def _mid_body(a0_ref, a1_ref, xws_ref, d0_ref, d1_ref, b_ref, w_ref, o_ref):
    dis = _dis(d0_ref[...], d1_ref[...])
    h = (a0_ref[...] + a1_ref[...] + xws_ref[...]) * dis + b_ref[...]
    h = jnp.maximum(h, 0.0)
    o_ref[...] = jnp.dot(h, w_ref[...], precision=_PREC,
                         preferred_element_type=jnp.float32) * dis


def _tc_mid(a0, a1, xws, d0, d1, b1, W2):
    return pl.pallas_call(
        _mid_body,
        grid=(_GRID,),
        in_specs=[
            pl.BlockSpec((_RB, H), lambda i: (i, 0)),
            pl.BlockSpec((_RB, H), lambda i: (i, 0)),
            pl.BlockSpec((_RB, H), lambda i: (i, 0)),
            pl.BlockSpec((_RB, 1), lambda i: (i, 0)),
            pl.BlockSpec((_RB, 1), lambda i: (i, 0)),
            pl.BlockSpec((1, H), lambda i: (0, 0)),
            pl.BlockSpec((H, H), lambda i: (0, 0)),
        ],
        out_specs=pl.BlockSpec((_RB, H), lambda i: (i, 0)),
        out_shape=jax.ShapeDtypeStruct((N, H), jnp.float32),
    )(a0, a1, xws, d0, d1, b1, W2)


def _head_body(a0_ref, a1_ref, xws_ref, d0_ref, d1_ref, b_ref, wc_ref, bc_ref,
               emb_ref, lp_ref):
    dis = _dis(d0_ref[...], d1_ref[...])
    emb = (a0_ref[...] + a1_ref[...] + xws_ref[...]) * dis + b_ref[...]
    emb = jnp.maximum(emb, 0.0)
    emb_ref[...] = emb
    logits = jnp.dot(emb, wc_ref[...], precision=_PREC,
                     preferred_element_type=jnp.float32) + bc_ref[...]
    m = jnp.max(logits, axis=1, keepdims=True)
    lse = jnp.log(jnp.sum(jnp.exp(logits - m), axis=1, keepdims=True)) + m
    lp_ref[...] = logits - lse


def _tc_head(a0, a1, xws, d0, d1, b2, Wc, bc):
    return pl.pallas_call(
        _head_body,
        grid=(_GRID,),
        in_specs=[
            pl.BlockSpec((_RB, H), lambda i: (i, 0)),
            pl.BlockSpec((_RB, H), lambda i: (i, 0)),
            pl.BlockSpec((_RB, H), lambda i: (i, 0)),
            pl.BlockSpec((_RB, 1), lambda i: (i, 0)),
            pl.BlockSpec((_RB, 1), lambda i: (i, 0)),
            pl.BlockSpec((1, H), lambda i: (0, 0)),
            pl.BlockSpec((H, C), lambda i: (0, 0)),
            pl.BlockSpec((1, C), lambda i: (0, 0)),
        ],
        out_specs=[
            pl.BlockSpec((_RB, H), lambda i: (i, 0)),
            pl.BlockSpec((_RB, C), lambda i: (i, 0)),
        ],
        out_shape=[
            jax.ShapeDtypeStruct((N, H), jnp.float32),
            jax.ShapeDtypeStruct((N, C), jnp.float32),
        ],
    )(a0, a1, xws, d0, d1, b2, Wc, bc)


def kernel(x, adj, W1, b1, W2, b2, Wc, bc):
    adj = adj.astype(jnp.int32)
    src = adj[0]
    dst = adj[1]
    z640 = jnp.zeros((640,), jnp.float32)
    zrows = jnp.zeros((ZR, H), jnp.float32)
    b1r = b1.reshape(1, H)
    b2r = b2.reshape(1, H)
    bcr = bc.reshape(1, C)

    degp = _sc_degree(dst, z640)
    d0 = degp[0, :N].reshape(N, 1)
    d1 = degp[1, :N].reshape(N, 1)

    xw1s = _tc_mm_scale(x, W1, d0, d1)
    accp1 = _sc_agg(xw1s, src, dst, zrows)
    xw2s = _tc_mid(accp1[0, :N], accp1[1, :N], xw1s, d0, d1, b1r, W2)
    accp2 = _sc_agg(xw2s, src, dst, zrows)
    emb, logp = _tc_head(accp2[0, :N], accp2[1, :N], xw2s, d0, d1, b2r, Wc, bcr)
    return (emb, logp)
